# Initial kernel scaffold; baseline (speedup 1.0000x reference)
import jax
import jax.numpy as jnp
from jax.experimental import pallas as pl


def kernel(x, edge_attr, params, edge_index, batch):
    raise NotImplementedError("write your pallas kernel here")


# speedup vs baseline: 2.4333x; 2.4333x over previous
import functools

import jax
import jax.numpy as jnp
from jax import lax
from jax.experimental import pallas as pl
from jax.experimental.pallas import tpu as pltpu
from jax.experimental.pallas import tpu_sc as plsc

N = 50000
E = 800000
NODE_DIM = 92
EDGE_DIM = 41
H = 128
G = 256
L = 3

BN_NODE = 2000
BE = 4000
NB_NODE = N // BN_NODE
NB_EDGE = E // BE

NW = 32
EPT = E // NW
KG = 200
BS = 200
F32 = jnp.float32


def _sp(x):
    return jnp.maximum(x, 0.0) + jnp.log1p(jnp.exp(-jnp.abs(x)))


def _fold_body(We2_ref, C_ref, be2_ref, bn1_ref, W2C_ref, bias2_ref):
    W2C_ref[...] = jnp.dot(We2_ref[...], C_ref[...], preferred_element_type=F32)
    bias2_ref[...] = jnp.dot(be2_ref[...], C_ref[...], preferred_element_type=F32) + bn1_ref[...]


_fold = pl.pallas_call(
    _fold_body,
    out_shape=(jax.ShapeDtypeStruct((H, H), F32), jax.ShapeDtypeStruct((1, H), F32)),
)


def _emb_body(x_ref, W_ref, b_ref, o_ref):
    o_ref[...] = jnp.dot(x_ref[...], W_ref[...], preferred_element_type=F32) + b_ref[...]


_emb = pl.pallas_call(
    _emb_body,
    grid=(NB_NODE,),
    in_specs=[
        pl.BlockSpec((BN_NODE, NODE_DIM), lambda i: (i, 0)),
        pl.BlockSpec((NODE_DIM, H), lambda i: (0, 0)),
        pl.BlockSpec((1, H), lambda i: (0, 0)),
    ],
    out_specs=pl.BlockSpec((BN_NODE, H), lambda i: (i, 0)),
    out_shape=jax.ShapeDtypeStruct((N, H), F32),
)


def _nodemm_body(h_ref, A_ref, B_ref, hA_ref, hB_ref):
    h = h_ref[...]
    hA_ref[...] = jnp.dot(h, A_ref[...], preferred_element_type=F32)
    hB_ref[...] = jnp.dot(h, B_ref[...], preferred_element_type=F32)


_nodemm = pl.pallas_call(
    _nodemm_body,
    grid=(NB_NODE,),
    in_specs=[
        pl.BlockSpec((BN_NODE, H), lambda i: (i, 0)),
        pl.BlockSpec((H, H), lambda i: (0, 0)),
        pl.BlockSpec((H, H), lambda i: (0, 0)),
    ],
    out_specs=(
        pl.BlockSpec((BN_NODE, H), lambda i: (i, 0)),
        pl.BlockSpec((BN_NODE, H), lambda i: (i, 0)),
    ),
    out_shape=(jax.ShapeDtypeStruct((N, H), F32), jax.ShapeDtypeStruct((N, H), F32)),
)


def _edge_body(ea_ref, gA_ref, gB_ref, We1_ref, be1_ref, W2C_ref, bias2_ref,
               Wn2_ref, bn2_ref, z_ref):
    u = jnp.dot(ea_ref[...], We1_ref[...], preferred_element_type=F32) + be1_ref[...]
    t = jnp.dot(_sp(u), W2C_ref[...], preferred_element_type=F32)
    pre = gA_ref[...] + gB_ref[...] + t + bias2_ref[...]
    z_ref[...] = jnp.dot(_sp(pre), Wn2_ref[...], preferred_element_type=F32) + bn2_ref[...]


_edge = pl.pallas_call(
    _edge_body,
    grid=(NB_EDGE,),
    in_specs=[
        pl.BlockSpec((BE, EDGE_DIM), lambda i: (i, 0)),
        pl.BlockSpec((BE, H), lambda i: (i, 0)),
        pl.BlockSpec((BE, H), lambda i: (i, 0)),
        pl.BlockSpec((EDGE_DIM, H), lambda i: (0, 0)),
        pl.BlockSpec((1, H), lambda i: (0, 0)),
        pl.BlockSpec((H, H), lambda i: (0, 0)),
        pl.BlockSpec((1, H), lambda i: (0, 0)),
        pl.BlockSpec((H, H), lambda i: (0, 0)),
        pl.BlockSpec((1, H), lambda i: (0, 0)),
    ],
    out_specs=pl.BlockSpec((BE, H), lambda i: (i, 0)),
    out_shape=jax.ShapeDtypeStruct((E, H), F32),
)


def _bnsum_body(h_ref, a0, a1, a2, a3, a4, a5, a6, a7, hu_ref, sums_ref, acc_ref):
    i = pl.program_id(0)
    aggr = jnp.concatenate(
        [a0[...] + a4[...], a1[...] + a5[...], a2[...] + a6[...], a3[...] + a7[...]],
        axis=-1)
    hu = h_ref[...] + aggr
    hu_ref[...] = hu

    @pl.when(i == 0)
    def _():
        acc_ref[...] = jnp.zeros_like(acc_ref)

    acc_ref[0:1, :] += jnp.sum(hu, axis=0, keepdims=True)
    acc_ref[1:2, :] += jnp.sum(hu * hu, axis=0, keepdims=True)

    @pl.when(i == NB_NODE - 1)
    def _():
        sums_ref[...] = acc_ref[...]


_bnsum = pl.pallas_call(
    _bnsum_body,
    grid=(NB_NODE,),
    in_specs=[pl.BlockSpec((BN_NODE, H), lambda i: (i, 0))] +
             [pl.BlockSpec((BN_NODE, 32), lambda i: (i, 0))] * 8,
    out_specs=(
        pl.BlockSpec((BN_NODE, H), lambda i: (i, 0)),
        pl.BlockSpec((8, H), lambda i: (0, 0)),
    ),
    out_shape=(jax.ShapeDtypeStruct((N, H), F32), jax.ShapeDtypeStruct((8, H), F32)),
    scratch_shapes=[pltpu.VMEM((8, H), F32)],
)


def _bnapply_body(hu_ref, sums_ref, gamma_ref, beta_ref, o_ref):
    sums = sums_ref[...]
    mean = sums[0:1, :] / N
    var = sums[1:2, :] / N - mean * mean
    inv = lax.rsqrt(var + 1e-5)
    o_ref[...] = _sp((hu_ref[...] - mean) * inv * gamma_ref[...] + beta_ref[...])


_bnapply = pl.pallas_call(
    _bnapply_body,
    grid=(NB_NODE,),
    in_specs=[
        pl.BlockSpec((BN_NODE, H), lambda i: (i, 0)),
        pl.BlockSpec((8, H), lambda i: (0, 0)),
        pl.BlockSpec((1, H), lambda i: (0, 0)),
        pl.BlockSpec((1, H), lambda i: (0, 0)),
    ],
    out_specs=pl.BlockSpec((BN_NODE, H), lambda i: (i, 0)),
    out_shape=jax.ShapeDtypeStruct((N, H), F32),
)


def _pool_body(h_ref, b_ref, Wf0_ref, bf0_ref, Wf1_ref, bf1_ref, Wo_ref, bo_ref,
               out_ref, sum_acc, cnt_acc):
    i = pl.program_id(0)

    @pl.when(i == 0)
    def _():
        sum_acc[...] = jnp.zeros_like(sum_acc)
        cnt_acc[...] = jnp.zeros_like(cnt_acc)

    hb = h_ref[...]
    bids = b_ref[0, 0, :]
    onehot = (lax.broadcasted_iota(jnp.int32, (G, BN_NODE), 0) == bids[None, :]).astype(F32)
    sum_acc[...] += jnp.dot(onehot, hb, preferred_element_type=F32)
    cnt_acc[...] += jnp.broadcast_to(jnp.sum(onehot, axis=1, keepdims=True), (G, H))

    @pl.when(i == NB_NODE - 1)
    def _():
        xm = sum_acc[...] / jnp.maximum(cnt_acc[...], 1.0)
        g2 = jnp.concatenate([xm, xm], axis=-1)
        t = _sp(jnp.dot(g2, Wf0_ref[...], preferred_element_type=F32) + bf0_ref[...])
        t = _sp(jnp.dot(t, Wf1_ref[...], preferred_element_type=F32) + bf1_ref[...])
        out_ref[...] = jnp.dot(t, Wo_ref[...], preferred_element_type=F32) + bo_ref[...]


_pool = pl.pallas_call(
    _pool_body,
    grid=(NB_NODE,),
    in_specs=[
        pl.BlockSpec((BN_NODE, H), lambda i: (i, 0)),
        pl.BlockSpec((1, 1, BN_NODE), lambda i: (i, 0, 0)),
        pl.BlockSpec((2 * H, H), lambda i: (0, 0)),
        pl.BlockSpec((1, H), lambda i: (0, 0)),
        pl.BlockSpec((H, H), lambda i: (0, 0)),
        pl.BlockSpec((1, H), lambda i: (0, 0)),
        pl.BlockSpec((H, 1), lambda i: (0, 0)),
        pl.BlockSpec((1, 1), lambda i: (0, 0)),
    ],
    out_specs=pl.BlockSpec((G, 1), lambda i: (0, 0)),
    out_shape=jax.ShapeDtypeStruct((G, 1), F32),
    scratch_shapes=[pltpu.VMEM((G, H), F32), pltpu.VMEM((G, H), F32)],
)


_sc_mesh = plsc.VectorSubcoreMesh(core_axis_name="c", subcore_axis_name="s")


@functools.partial(
    pl.kernel,
    mesh=_sc_mesh,
    out_type=(jax.ShapeDtypeStruct((E, H), F32), jax.ShapeDtypeStruct((E, H), F32)),
    scratch_types=[
        pltpu.VMEM((KG,), jnp.int32),
        pltpu.VMEM((KG,), jnp.int32),
        pltpu.VMEM((KG, H), F32),
        pltpu.VMEM((KG, H), F32),
        pltpu.SemaphoreType.DMA,
        pltpu.SemaphoreType.DMA,
    ],
)
def _gather2(hA_hbm, hB_hbm, dst_hbm, src_hbm, gA_hbm, gB_hbm,
             idxd, idxs, bufA, bufB, semA, semB):
    c = lax.axis_index("c")
    s = lax.axis_index("s")
    wid = s * 2 + c
    base = wid * EPT

    def step(j, carry):
        off = base + j * KG
        pltpu.sync_copy(dst_hbm.at[pl.ds(off, KG)], idxd)
        pltpu.sync_copy(src_hbm.at[pl.ds(off, KG)], idxs)
        cpA = pltpu.async_copy(hA_hbm.at[idxd], bufA, semA)
        cpB = pltpu.async_copy(hB_hbm.at[idxs], bufB, semB)
        cpA.wait()
        cpB.wait()
        pltpu.sync_copy(bufA, gA_hbm.at[pl.ds(off, KG)])
        pltpu.sync_copy(bufB, gB_hbm.at[pl.ds(off, KG)])
        return carry

    lax.fori_loop(0, EPT // KG, step, 0)


@functools.partial(
    pl.kernel,
    mesh=_sc_mesh,
    compiler_params=pltpu.CompilerParams(use_tc_tiling_on_sc=False),
    out_type=jax.ShapeDtypeStruct((4, 2, N, 32), F32),
    scratch_types=[
        pltpu.VMEM((BS,), jnp.int32),
        pltpu.VMEM((BS, 32), F32),
        pltpu.VMEM_SHARED((N, 32), F32),
    ],
)
def _scatter(z_hbm, dst_hbm, zeros_hbm, aggr_hbm, idxb, zbuf, acc):
    c = lax.axis_index("c")
    s = lax.axis_index("s")
    wid = c * 16 + s
    base = wid * EPT

    for chunk in range(4):
        col0 = 32 * chunk

        @pl.when(s == 0)
        def _():
            pltpu.sync_copy(zeros_hbm, acc)

        plsc.subcore_barrier()

        def step(j, carry):
            off = base + j * BS
            pltpu.sync_copy(dst_hbm.at[pl.ds(off, BS)], idxb)
            pltpu.sync_copy(z_hbm.at[pl.ds(off, BS), pl.ds(col0, 32)], zbuf)
            pltpu.sync_copy(zbuf, acc.at[idxb], add=True)
            return carry

        lax.fori_loop(0, EPT // BS, step, 0)
        plsc.subcore_barrier()

        @pl.when(s == 0)
        def _():
            pltpu.sync_copy(acc, aggr_hbm.at[chunk, c])

        plsc.subcore_barrier()


@jax.jit
def _run(x, edge_attr, params, edge_index, batch):
    src = edge_index[0]
    dst = edge_index[1]
    zeros32 = jnp.zeros((N, 32), F32)
    batch3 = batch.reshape(NB_NODE, 1, BN_NODE)

    def row(b):
        return b.reshape(1, -1)

    h = _emb(x, params["emb"]["W"], row(params["emb"]["b"]))
    for i in range(L):
        cv = params["convs"][i]
        Wn1 = cv["n1"]["W"]
        A, B, C = Wn1[:H], Wn1[H:2 * H], Wn1[2 * H:]
        W2C, bias2 = _fold(cv["e2"]["W"], C, row(cv["e2"]["b"]), row(cv["n1"]["b"]))
        hA, hB = _nodemm(h, A, B)
        gA, gB = _gather2(hA, hB, dst, src)
        z = _edge(edge_attr, gA, gB, cv["e1"]["W"], row(cv["e1"]["b"]), W2C, bias2,
                  cv["n2"]["W"], row(cv["n2"]["b"]))
        aggr = _scatter(z, dst, zeros32)
        parts = [aggr[q, p] for p in range(2) for q in range(4)]
        bn = params["bns"][i]
        hu, sums = _bnsum(h, *parts)
        h = _bnapply(hu, sums, row(bn["gamma"]), row(bn["beta"]))

    fc0, fc1 = params["fcs"]
    out = _pool(h, batch3, fc0["W"], row(fc0["b"]), fc1["W"], row(fc1["b"]),
                params["out"]["W"], params["out"]["b"].reshape(1, 1))
    return out


def kernel(x, edge_attr, params, edge_index, batch):
    return _run(x, edge_attr, params, edge_index, batch)
